# Initial kernel scaffold; baseline (speedup 1.0000x reference)
import functools

import jax
import jax.numpy as jnp
from jax import lax
from jax.experimental import pallas as pl
from jax.experimental.pallas import tpu as pltpu
from jax.experimental.pallas import tpu_sc as plsc

BATCH = 16384
DATA_SIZE = 6
NUM_CLASSES = 10
OUT_D = 128
NUM_TABLES = 5

_info = plsc.get_sparse_core_info()
NC, NS, L = _info.num_cores, _info.num_subcores, _info.num_lanes
NW = NC * NS
BPW = BATCH // NW
GROUPS = BPW // L
ROW_W = DATA_SIZE * NUM_CLASSES


def _sc_body(x_hbm, e1, e2, e3, e4, e5, out_hbm,
             xs, i1, i2, i3, i4, i5, rows, zbuf, sem):
    tables = (e1, e2, e3, e4, e5)
    idx_refs = (i1, i2, i3, i4, i5)
    wid = lax.axis_index("s") * NC + lax.axis_index("c")
    base = wid * BPW

    pltpu.sync_copy(x_hbm.at[pl.ds(base * ROW_W, BPW * ROW_W)], xs)

    lanes = lax.iota(jnp.int32, L)

    def group_body(g, carry):
        gbase = lanes * ROW_W + g * (L * ROW_W)
        acc = jnp.zeros((L,), jnp.int32)
        pw = 1
        for d in range(NUM_TABLES):
            best = plsc.load_gather(xs, [gbase + d * NUM_CLASSES])
            sidx = jnp.zeros((L,), jnp.int32)
            for c in range(1, NUM_CLASSES):
                v = plsc.load_gather(xs, [gbase + (d * NUM_CLASSES + c)])
                m = v > best
                best = jnp.where(m, v, best)
                sidx = jnp.where(m, c, sidx)
            acc = acc + sidx * pw
            pw *= NUM_CLASSES
            idx_refs[d][pl.ds(g * L, L)] = acc
        return carry

    lax.fori_loop(0, GROUPS, group_body, 0)

    for d in range(NUM_TABLES):
        pltpu.async_copy(tables[d].at[idx_refs[d]], rows, sem).wait()
        pltpu.sync_copy(rows, out_hbm.at[pl.ds(base, BPW), d + 1])

    def zrow(r, carry):
        for cc in range(OUT_D // L):
            zbuf[r, pl.ds(cc * L, L)] = jnp.zeros((L,), jnp.float32)
        return carry

    lax.fori_loop(0, OUT_D, zrow, 0)
    for j in range(BPW // OUT_D):
        pltpu.sync_copy(zbuf, out_hbm.at[pl.ds(base + j * OUT_D, OUT_D), 0])


@functools.partial(
    pl.kernel,
    out_type=jax.ShapeDtypeStruct((BATCH, DATA_SIZE, OUT_D), jnp.float32),
    mesh=plsc.VectorSubcoreMesh(core_axis_name="c", subcore_axis_name="s"),
    scratch_types=[
        pltpu.VMEM((BPW * ROW_W,), jnp.float32),
        pltpu.VMEM((BPW,), jnp.int32),
        pltpu.VMEM((BPW,), jnp.int32),
        pltpu.VMEM((BPW,), jnp.int32),
        pltpu.VMEM((BPW,), jnp.int32),
        pltpu.VMEM((BPW,), jnp.int32),
        pltpu.VMEM((BPW, OUT_D), jnp.float32),
        pltpu.VMEM((OUT_D, OUT_D), jnp.float32),
        pltpu.SemaphoreType.DMA,
    ],
)
def _sc_kernel(x_hbm, e1, e2, e3, e4, e5, out_hbm, *scratch):
    _sc_body(x_hbm, e1, e2, e3, e4, e5, out_hbm, *scratch)


def kernel(inputs, emb1, emb2, emb3, emb4, emb5):
    x = inputs.reshape(BATCH * ROW_W)
    return _sc_kernel(x, emb1, emb2, emb3, emb4, emb5)


# speedup vs baseline: 1.4810x; 1.4810x over previous
import functools

import jax
import jax.numpy as jnp
from jax import lax
from jax.experimental import pallas as pl
from jax.experimental.pallas import tpu as pltpu
from jax.experimental.pallas import tpu_sc as plsc

BATCH = 16384
DATA_SIZE = 6
NUM_CLASSES = 10
OUT_D = 128
NUM_TABLES = 5

_info = plsc.get_sparse_core_info()
NS, L = _info.num_subcores, _info.num_lanes
NC = 1
NW = NC * NS
BPW = BATCH // NW
GROUPS = BPW // L
ROW_W = DATA_SIZE * NUM_CLASSES

NBUF = 4
CH = 128
NSEC = BPW // CH
NCH = NUM_TABLES * NSEC
HALF = BPW // 2
GPS = CH // L
TL_OFF = (0, NUM_CLASSES, NUM_CLASSES + NUM_CLASSES ** 2)
TL_ROWS = NUM_CLASSES + NUM_CLASSES ** 2 + NUM_CLASSES ** 3
N_LOCAL = 3


def _sc_body(x_hbm, e1, e2, e3, e4, e5, out_hbm,
             xs, tl, i1, i2, i3, i4, i5, r0, r1, r2, r3, zbuf,
             g0, g1, g2, g3, w0, w1, w2, w3, zsem):
    tables = (e1, e2, e3, e4, e5)
    idx_refs = (i1, i2, i3, i4, i5)
    bufs = (r0, r1, r2, r3)
    gsems = (g0, g1, g2, g3)
    wsems = (w0, w1, w2, w3)
    wid = lax.axis_index("s")
    base = wid * BPW

    def stage_half(half):
        pltpu.sync_copy(
            x_hbm.at[pl.ds((base + half * HALF) * ROW_W, HALF * ROW_W)], xs)

    stage_half(0)

    @pl.when(lax.axis_index("s") == 0)
    def _stage_tables():
        pltpu.sync_copy(e1, tl.at[pl.ds(TL_OFF[0], NUM_CLASSES)])
        pltpu.sync_copy(e2, tl.at[pl.ds(TL_OFF[1], NUM_CLASSES ** 2)])
        pltpu.sync_copy(e3, tl.at[pl.ds(TL_OFF[2], NUM_CLASSES ** 3)])

    plsc.subcore_barrier()

    def zrow(r, carry):
        for cc in range(OUT_D // L):
            zbuf[r, pl.ds(cc * L, L)] = jnp.zeros((L,), jnp.float32)
        return carry

    lax.fori_loop(0, CH, zrow, 0)
    zcopies = [
        pltpu.async_copy(zbuf, out_hbm.at[pl.ds(base + j * CH, CH), 0], zsem)
        for j in range(BPW // CH)
    ]

    lanes = lax.iota(jnp.int32, L)

    def section_argmax(h):
        goff = (h % (NSEC // 2)) * GPS
        soff = h * GPS

        def group_body(gl, carry):
            gbase = lanes * ROW_W + (gl + goff) * (L * ROW_W)
            acc = jnp.zeros((L,), jnp.int32)
            pw = 1
            for d in range(NUM_TABLES):
                best = plsc.load_gather(xs, [gbase + d * NUM_CLASSES])
                sidx = jnp.zeros((L,), jnp.int32)
                for c in range(1, NUM_CLASSES):
                    v = plsc.load_gather(xs, [gbase + (d * NUM_CLASSES + c)])
                    m = v > best
                    best = jnp.where(m, v, best)
                    sidx = jnp.where(m, c, sidx)
                acc = acc + sidx * pw
                pw *= NUM_CLASSES
                off = TL_OFF[d] if d < N_LOCAL else 0
                idx_refs[d][pl.ds((soff + gl) * L, L)] = (
                    acc + off if off else acc)
            return carry

        lax.fori_loop(0, GPS, group_body, 0)

    def start_gather(k):
        h, d = divmod(k, NUM_TABLES)
        src = tl if d < N_LOCAL else tables[d]
        return pltpu.async_copy(
            src.at[idx_refs[d].at[pl.ds(h * CH, CH)]],
            bufs[k % NBUF], gsems[k % NBUF])

    def advance_argmax(done_h, need_h):
        while done_h <= need_h:
            if done_h == NSEC // 2:
                stage_half(1)
            section_argmax(done_h)
            done_h += 1
        return done_h

    done_h = advance_argmax(0, 0)
    gops = [None] * NCH
    writes = [None] * NBUF
    for k in range(min(NBUF - 1, NCH)):
        done_h = advance_argmax(done_h, k // NUM_TABLES)
        gops[k] = start_gather(k)
    for k in range(NCH):
        b = k % NBUF
        j = k + NBUF - 1
        if j < NCH:
            done_h = advance_argmax(done_h, j // NUM_TABLES)
            bb = j % NBUF
            if writes[bb] is not None:
                writes[bb].wait()
            gops[j] = start_gather(j)
        gops[k].wait()
        h, d = divmod(k, NUM_TABLES)
        writes[b] = pltpu.async_copy(
            bufs[b], out_hbm.at[pl.ds(base + h * CH, CH), d + 1], wsems[b])
    for wop in writes:
        wop.wait()
    for zop in zcopies:
        zop.wait()


@functools.partial(
    pl.kernel,
    out_type=jax.ShapeDtypeStruct((BATCH, DATA_SIZE, OUT_D), jnp.float32),
    mesh=plsc.VectorSubcoreMesh(
        core_axis_name="c", subcore_axis_name="s", num_cores=1),
    compiler_params=pltpu.CompilerParams(needs_layout_passes=False),
    scratch_types=[
        pltpu.VMEM((HALF * ROW_W,), jnp.float32),
        pltpu.VMEM_SHARED((TL_ROWS, OUT_D), jnp.float32),
        pltpu.VMEM((BPW,), jnp.int32),
        pltpu.VMEM((BPW,), jnp.int32),
        pltpu.VMEM((BPW,), jnp.int32),
        pltpu.VMEM((BPW,), jnp.int32),
        pltpu.VMEM((BPW,), jnp.int32),
        pltpu.VMEM((CH, OUT_D), jnp.float32),
        pltpu.VMEM((CH, OUT_D), jnp.float32),
        pltpu.VMEM((CH, OUT_D), jnp.float32),
        pltpu.VMEM((CH, OUT_D), jnp.float32),
        pltpu.VMEM((CH, OUT_D), jnp.float32),
        pltpu.SemaphoreType.DMA,
        pltpu.SemaphoreType.DMA,
        pltpu.SemaphoreType.DMA,
        pltpu.SemaphoreType.DMA,
        pltpu.SemaphoreType.DMA,
        pltpu.SemaphoreType.DMA,
        pltpu.SemaphoreType.DMA,
        pltpu.SemaphoreType.DMA,
        pltpu.SemaphoreType.DMA,
    ],
)
def _sc_kernel(x_hbm, e1, e2, e3, e4, e5, out_hbm, *scratch):
    _sc_body(x_hbm, e1, e2, e3, e4, e5, out_hbm, *scratch)


def kernel(inputs, emb1, emb2, emb3, emb4, emb5):
    x = inputs.reshape(BATCH * ROW_W)
    return _sc_kernel(x, emb1, emb2, emb3, emb4, emb5)
